# baseline (speedup 1.0000x reference)
import functools

import numpy as np
import jax
import jax.numpy as jnp
from jax.experimental import pallas as pl
from jax.experimental.pallas import tpu as pltpu

H = 128
NR = 6
NS = 7
NB = 8
CUTOFF = 5.0
_F32 = jnp.float32


def _tile(n, cap=2048):
    c = min(cap, n)
    c -= c % 8
    while c >= 8:
        if n % c == 0:
            return c
        c -= 8
    return n


def _swish(v):
    return v * jax.nn.sigmoid(v)


def _dot(a, b):
    return jnp.dot(a, b, preferred_element_type=_F32)


def _iota_row(k):
    return jax.lax.broadcasted_iota(jnp.int32, (1, k), 1).astype(_F32)


def _rows(bsz, width):
    return pl.BlockSpec((bsz, width), lambda i, *_: (i, 0))


def _bcast(shape):
    nd = len(shape)
    return pl.BlockSpec(shape, lambda i, *_, _nd=nd: (0,) * _nd)


def _embed_body(ps_r, pd_r, zs_r, zd_r, emb_r, wr_r, w1_r, w2_r, w3_r, b_r,
                wg_r, dvec_o, d_o, rbf_o, x_o, g_o):
    dvec = pd_r[...] - ps_r[...]
    d = jnp.sqrt(jnp.sum(dvec * dvec, axis=1, keepdims=True) + 1e-12)
    n = _iota_row(NR) + 1.0
    rbf = jnp.exp(-(d / CUTOFF) ** 2) * jnp.sin(d * (n * (np.pi / CUTOFF))) / (d + 1e-6)
    hrbf = _swish(_dot(rbf, wr_r[...]))
    zcols = jax.lax.broadcasted_iota(jnp.int32, (zs_r.shape[0], 96), 1).astype(_F32)
    es = _dot(jnp.where(zs_r[...] == zcols, 1.0, 0.0), emb_r[...])
    ed = _dot(jnp.where(zd_r[...] == zcols, 1.0, 0.0), emb_r[...])
    x = _swish(_dot(es, w1_r[...]) + _dot(ed, w2_r[...])
               + _dot(hrbf, w3_r[...]) + b_r[...])
    dvec_o[...] = dvec
    d_o[...] = d
    rbf_o[...] = rbf
    x_o[...] = x
    g_o[...] = _dot(rbf, wg_r[...]) * x


def _trigeom_body(v1_r, v2_r, dk_r, sbf_o):
    v1 = v1_r[...]
    v2 = -v2_r[...]
    num = jnp.sum(v1 * v2, axis=1, keepdims=True)
    n1 = jnp.sqrt(jnp.sum(v1 * v1, axis=1, keepdims=True))
    n2 = jnp.sqrt(jnp.sum(v2 * v2, axis=1, keepdims=True))
    c = jnp.clip(num / (n1 * n2 + 1e-6), -0.999999, 0.999999)
    dk = dk_r[...]
    n = _iota_row(NR) + 1.0
    rad = jnp.sin(dk * (n * (np.pi / CUTOFF))) / (dk + 1e-6)
    parts = [rad]
    t_prev = jnp.ones_like(c)
    t_cur = c
    parts.append(rad * t_cur)
    for _ in range(2, NS):
        t_prev, t_cur = t_cur, 2.0 * c * t_cur - t_prev
        parts.append(rad * t_cur)
    sbf_o[...] = jnp.concatenate(parts, axis=1)


def _stage_a_body(x_r, rbf_r, wlr_r, wkj_r, bkj_r, wji_r, bji_r,
                  xji_o, xkj_o):
    x = x_r[...]
    rbf_h = _dot(rbf_r[...], wlr_r[...])
    xji_o[...] = _swish(_dot(x, wji_r[...]) + bji_r[...])
    xkj_o[...] = _swish(_dot(x, wkj_r[...]) + bkj_r[...]) * rbf_h


def _tri_body(sbf_r, xk_r, ws_r, wbil_r, y_o):
    sbf_h = _dot(sbf_r[...], ws_r[...])
    xk = xk_r[...]
    acc = sbf_h[:, 0:1] * _dot(xk, wbil_r[0])
    for b in range(1, NB):
        acc = acc + sbf_h[:, b:b + 1] * _dot(xk, wbil_r[b])
    y_o[...] = acc


def _stage_b_scatter_body(starts_r, y_r, lid_r, xji_r, x_r, rbf_r, wbs_r,
                          bbs_r, wl_r, bl_r, wa1_r, ba1_r, wa2_r, ba2_r,
                          wg_r, x_o, g_o, acc_r, *, ba, ch):
    i = pl.program_id(0)
    s = starts_r[i]
    e = starts_r[i + 1]
    s0 = (s // ch) * ch
    nch = (e - s0 + (ch - 1)) // ch
    acc_r[...] = jnp.zeros_like(acc_r)

    col = jax.lax.broadcasted_iota(jnp.int32, (ch, ba), 1).astype(_F32)
    row = jax.lax.broadcasted_iota(jnp.int32, (ch, 1), 0)

    def body(k, carry):
        base = s0 + k * ch
        yc = y_r[pl.ds(base, ch), :]
        lc = lid_r[pl.ds(base, ch), :]
        gidx = row + base
        oh = jnp.where((lc == col) & (gidx >= s) & (gidx < e), 1.0, 0.0)
        acc_r[...] += jax.lax.dot_general(
            oh, yc, (((0,), (0,)), ((), ())), preferred_element_type=_F32)
        return carry

    jax.lax.fori_loop(0, nch, body, 0)

    h = xji_r[...] + acc_r[...]
    h = h + _swish(_dot(h, wbs_r[...]) + bbs_r[...])
    h = _swish(_dot(h, wl_r[...]) + bl_r[...]) + x_r[...]
    h = h + _swish(_dot(h, wa1_r[...]) + ba1_r[...])
    h = h + _swish(_dot(h, wa2_r[...]) + ba2_r[...])
    x_o[...] = h
    g_o[...] = _dot(rbf_r[...], wg_r[...]) * h


def _stage_b_body(xji_r, agg_r, x_r, rbf_r, wbs_r, bbs_r, wl_r, bl_r,
                  wa1_r, ba1_r, wa2_r, ba2_r, wg_r, x_o, g_o):
    h = xji_r[...] + agg_r[...]
    h = h + _swish(_dot(h, wbs_r[...]) + bbs_r[...])
    h = _swish(_dot(h, wl_r[...]) + bl_r[...]) + x_r[...]
    h = h + _swish(_dot(h, wa1_r[...]) + ba1_r[...])
    h = h + _swish(_dot(h, wa2_r[...]) + ba2_r[...])
    x_o[...] = h
    g_o[...] = _dot(rbf_r[...], wg_r[...]) * h


def _out_node_body(node_r, w1_r, b1_r, w2_r, b2_r, w3_r, b3_r, wo_r, out_o):
    nd = _swish(_dot(node_r[...], w1_r[...]) + b1_r[...])
    nd = _swish(_dot(nd, w2_r[...]) + b2_r[...])
    nd = _swish(_dot(nd, w3_r[...]) + b3_r[...])
    s = jnp.sum(_dot(nd, wo_r[...]))
    i = pl.program_id(0)

    @pl.when(i == 0)
    def _():
        out_o[...] = jnp.reshape(s, (1, 1))

    @pl.when(i != 0)
    def _():
        out_o[...] = out_o[...] + jnp.reshape(s, (1, 1))


def _sds(shape):
    return jax.ShapeDtypeStruct(shape, _F32)


def kernel(z, pos, edge_index, idx_kj, idx_ji, params):
    E = edge_index.shape[1]
    T = idx_kj.shape[0]
    N = pos.shape[0]
    be = _tile(E)
    bt = _tile(T)
    bn = _tile(N, 1024)
    ge, gt, gn = E // be, T // bt, N // bn

    src = edge_index[0]
    dst = edge_index[1]
    ps = jnp.take(pos, src, axis=0)
    pd = jnp.take(pos, dst, axis=0)
    emb_pad = jnp.concatenate(
        [params['emb'], jnp.zeros((96 - params['emb'].shape[0], H), _F32)],
        axis=0)
    zs = jnp.take(z, src, axis=0).astype(_F32).reshape(E, 1)
    zd = jnp.take(z, dst, axis=0).astype(_F32).reshape(E, 1)
    wlin = params['emb_lin_W']
    w1, w2, w3 = wlin[:H], wlin[H:2 * H], wlin[2 * H:]
    b_emb = params['emb_lin_b'].reshape(1, H)
    outs = params['outs']
    blocks = params['blocks']

    dvec, d, rbf, x, g = pl.pallas_call(
        _embed_body,
        grid=(ge,),
        in_specs=[_rows(be, 3), _rows(be, 3), _rows(be, 1), _rows(be, 1),
                  _bcast((96, H)), _bcast((NR, H)), _bcast((H, H)),
                  _bcast((H, H)), _bcast((H, H)), _bcast((1, H)),
                  _bcast((NR, H))],
        out_specs=[_rows(be, 3), _rows(be, 1), _rows(be, NR), _rows(be, H),
                   _rows(be, H)],
        out_shape=[_sds((E, 3)), _sds((E, 1)), _sds((E, NR)), _sds((E, H)),
                   _sds((E, H))],
    )(ps, pd, zs, zd, emb_pad, params['emb_rbf'], w1, w2, w3, b_emb,
      outs[0]['lin_rbf'])

    ba = _tile(E, 512)
    gb = E // ba
    ch = 128
    p_tri = jnp.argsort(idx_ji)
    sji = jnp.take(idx_ji, p_tri)
    skj = jnp.take(idx_kj, p_tri)
    starts = jnp.searchsorted(sji, jnp.arange(0, E + 1, ba)).astype(jnp.int32)
    lid_pad = jnp.concatenate(
        [(sji % ba).astype(_F32).reshape(-1, 1),
         jnp.full((ch, 1), -1.0, _F32)], axis=0)
    tp = T + ch

    v1 = jnp.take(dvec, sji, axis=0)
    v2 = jnp.take(dvec, skj, axis=0)
    dk = jnp.take(d, skj, axis=0)
    sbf = pl.pallas_call(
        _trigeom_body,
        grid=(gt,),
        in_specs=[_rows(bt, 3), _rows(bt, 3), _rows(bt, 1)],
        out_specs=_rows(bt, NS * NR),
        out_shape=_sds((T, NS * NR)),
    )(v1, v2, dk)

    def run_out(po, g_edge):
        node = jax.ops.segment_sum(g_edge, dst, num_segments=N)
        return pl.pallas_call(
            _out_node_body,
            grid=(gn,),
            in_specs=[_rows(bn, H), _bcast((H, H)), _bcast((1, H)),
                      _bcast((H, H)), _bcast((1, H)), _bcast((H, H)),
                      _bcast((1, H)), _bcast((H, 1))],
            out_specs=_bcast((1, 1)),
            out_shape=_sds((1, 1)),
        )(node, po['l1_W'], po['l1_b'].reshape(1, H), po['l2_W'],
          po['l2_b'].reshape(1, H), po['l3_W'], po['l3_b'].reshape(1, H),
          po['out_W'])

    P = run_out(outs[0], g)

    for bi, pb in enumerate(blocks):
        xji, xkj = pl.pallas_call(
            _stage_a_body,
            grid=(ge,),
            in_specs=[_rows(be, H), _rows(be, NR), _bcast((NR, H)),
                      _bcast((H, H)), _bcast((1, H)), _bcast((H, H)),
                      _bcast((1, H))],
            out_specs=[_rows(be, H), _rows(be, H)],
            out_shape=[_sds((E, H)), _sds((E, H))],
        )(x, rbf, pb['lin_rbf'], pb['kj_W'], pb['kj_b'].reshape(1, H),
          pb['ji_W'], pb['ji_b'].reshape(1, H))

        xk = jnp.take(xkj, skj, axis=0)
        y = pl.pallas_call(
            _tri_body,
            grid=(gt,),
            in_specs=[_rows(bt, NS * NR), _rows(bt, H),
                      _bcast((NS * NR, NB)), _bcast((NB, H, H))],
            out_specs=_rows(bt, H),
            out_shape=_sds((T, H)),
        )(sbf, xk, pb['lin_sbf'], pb['Wbil'])

        y_pad = jnp.concatenate([y, jnp.zeros((ch, H), _F32)], axis=0)
        x, g = pl.pallas_call(
            functools.partial(_stage_b_scatter_body, ba=ba, ch=ch),
            grid_spec=pltpu.PrefetchScalarGridSpec(
                num_scalar_prefetch=1,
                grid=(gb,),
                in_specs=[_bcast((tp, H)), _bcast((tp, 1)), _rows(ba, H),
                          _rows(ba, H), _rows(ba, NR), _bcast((H, H)),
                          _bcast((1, H)), _bcast((H, H)), _bcast((1, H)),
                          _bcast((H, H)), _bcast((1, H)), _bcast((H, H)),
                          _bcast((1, H)), _bcast((NR, H))],
                out_specs=[_rows(ba, H), _rows(ba, H)],
                scratch_shapes=[pltpu.VMEM((ba, H), _F32)],
            ),
            out_shape=[_sds((E, H)), _sds((E, H))],
        )(starts, y_pad, lid_pad, xji, x, rbf, pb['bs_W'],
          pb['bs_b'].reshape(1, H), pb['lin_W'], pb['lin_b'].reshape(1, H),
          pb['as1_W'], pb['as1_b'].reshape(1, H), pb['as2_W'],
          pb['as2_b'].reshape(1, H), outs[bi + 1]['lin_rbf'])

        P = P + run_out(outs[bi + 1], g)

    return P.reshape((1,))


# speedup vs baseline: 1.0927x; 1.0927x over previous
import functools

import numpy as np
import jax
import jax.numpy as jnp
from jax.experimental import pallas as pl
from jax.experimental.pallas import tpu as pltpu

H = 128
NR = 6
NS = 7
NB = 8
CUTOFF = 5.0
_F32 = jnp.float32


def _tile(n, cap=2048):
    c = min(cap, n)
    c -= c % 8
    while c >= 8:
        if n % c == 0:
            return c
        c -= 8
    return n


def _swish(v):
    return v * jax.nn.sigmoid(v)


def _dot(a, b):
    return jnp.dot(a, b, preferred_element_type=_F32)


def _iota_row(k):
    return jax.lax.broadcasted_iota(jnp.int32, (1, k), 1).astype(_F32)


def _rows(bsz, width):
    return pl.BlockSpec((bsz, width), lambda i, *_: (i, 0))


def _bcast(shape):
    nd = len(shape)
    return pl.BlockSpec(shape, lambda i, *_, _nd=nd: (0,) * _nd)


def _embed_body(ps_r, pd_r, zs_r, zd_r, emb_r, wr_r, w1_r, w2_r, w3_r, b_r,
                wg_r, dvec_o, d_o, rbf_o, x_o, g_o):
    dvec = pd_r[...] - ps_r[...]
    d = jnp.sqrt(jnp.sum(dvec * dvec, axis=1, keepdims=True) + 1e-12)
    n = _iota_row(NR) + 1.0
    rbf = jnp.exp(-(d / CUTOFF) ** 2) * jnp.sin(d * (n * (np.pi / CUTOFF))) / (d + 1e-6)
    hrbf = _swish(_dot(rbf, wr_r[...]))
    zcols = jax.lax.broadcasted_iota(jnp.int32, (zs_r.shape[0], 96), 1).astype(_F32)
    es = _dot(jnp.where(zs_r[...] == zcols, 1.0, 0.0), emb_r[...])
    ed = _dot(jnp.where(zd_r[...] == zcols, 1.0, 0.0), emb_r[...])
    x = _swish(_dot(es, w1_r[...]) + _dot(ed, w2_r[...])
               + _dot(hrbf, w3_r[...]) + b_r[...])
    dvec_o[...] = dvec
    d_o[...] = d
    rbf_o[...] = rbf
    x_o[...] = x
    g_o[...] = _dot(rbf, wg_r[...]) * x


def _trigeom_body(v1_r, v2_r, dk_r, sbf_o):
    v1 = v1_r[...]
    v2 = -v2_r[...]
    num = jnp.sum(v1 * v2, axis=1, keepdims=True)
    n1 = jnp.sqrt(jnp.sum(v1 * v1, axis=1, keepdims=True))
    n2 = jnp.sqrt(jnp.sum(v2 * v2, axis=1, keepdims=True))
    c = jnp.clip(num / (n1 * n2 + 1e-6), -0.999999, 0.999999)
    dk = dk_r[...]
    n = _iota_row(NR) + 1.0
    rad = jnp.sin(dk * (n * (np.pi / CUTOFF))) / (dk + 1e-6)
    parts = [rad]
    t_prev = jnp.ones_like(c)
    t_cur = c
    parts.append(rad * t_cur)
    for _ in range(2, NS):
        t_prev, t_cur = t_cur, 2.0 * c * t_cur - t_prev
        parts.append(rad * t_cur)
    sbf_o[...] = jnp.concatenate(parts, axis=1)


def _stage_a_body(x_r, rbf_r, wlr_r, wkj_r, bkj_r, wji_r, bji_r,
                  xji_o, xkj_o):
    x = x_r[...]
    rbf_h = _dot(rbf_r[...], wlr_r[...])
    xji_o[...] = _swish(_dot(x, wji_r[...]) + bji_r[...])
    xkj_o[...] = _swish(_dot(x, wkj_r[...]) + bkj_r[...]) * rbf_h


def _tri_body(sbf_r, xk_r, ws_r, wbil_r, y_o):
    sbf_h = _dot(sbf_r[...], ws_r[...])
    xk = xk_r[...]
    acc = sbf_h[:, 0:1] * _dot(xk, wbil_r[0])
    for b in range(1, NB):
        acc = acc + sbf_h[:, b:b + 1] * _dot(xk, wbil_r[b])
    y_o[...] = acc


def _stage_b_scatter_body(starts_r, y_r, lid_r, xji_r, x_r, rbf_r, wbs_r,
                          bbs_r, wl_r, bl_r, wa1_r, ba1_r, wa2_r, ba2_r,
                          wg_r, x_o, g_o, acc_r, *, ba, ch):
    i = pl.program_id(0)
    s = starts_r[i]
    e = starts_r[i + 1]
    s0 = (s // ch) * ch
    nch = (e - s0 + (ch - 1)) // ch
    acc_r[...] = jnp.zeros_like(acc_r)

    col = jax.lax.broadcasted_iota(jnp.int32, (ch, ba), 1).astype(_F32)
    row = jax.lax.broadcasted_iota(jnp.int32, (ch, 1), 0)

    def body(k, carry):
        base = s0 + k * ch
        yc = y_r[pl.ds(base, ch), :]
        lc = lid_r[pl.ds(base, ch), :]
        gidx = row + base
        oh = jnp.where((lc == col) & (gidx >= s) & (gidx < e), 1.0, 0.0)
        acc_r[...] += jax.lax.dot_general(
            oh, yc, (((0,), (0,)), ((), ())), preferred_element_type=_F32)
        return carry

    jax.lax.fori_loop(0, nch, body, 0)

    h = xji_r[...] + acc_r[...]
    h = h + _swish(_dot(h, wbs_r[...]) + bbs_r[...])
    h = _swish(_dot(h, wl_r[...]) + bl_r[...]) + x_r[...]
    h = h + _swish(_dot(h, wa1_r[...]) + ba1_r[...])
    h = h + _swish(_dot(h, wa2_r[...]) + ba2_r[...])
    x_o[...] = h
    g_o[...] = _dot(rbf_r[...], wg_r[...]) * h


def _stage_b_body(xji_r, agg_r, x_r, rbf_r, wbs_r, bbs_r, wl_r, bl_r,
                  wa1_r, ba1_r, wa2_r, ba2_r, wg_r, x_o, g_o):
    h = xji_r[...] + agg_r[...]
    h = h + _swish(_dot(h, wbs_r[...]) + bbs_r[...])
    h = _swish(_dot(h, wl_r[...]) + bl_r[...]) + x_r[...]
    h = h + _swish(_dot(h, wa1_r[...]) + ba1_r[...])
    h = h + _swish(_dot(h, wa2_r[...]) + ba2_r[...])
    x_o[...] = h
    g_o[...] = _dot(rbf_r[...], wg_r[...]) * h


def _out_node_body(node_r, w1_r, b1_r, w2_r, b2_r, w3_r, b3_r, wo_r, out_o):
    nd = _swish(_dot(node_r[...], w1_r[...]) + b1_r[...])
    nd = _swish(_dot(nd, w2_r[...]) + b2_r[...])
    nd = _swish(_dot(nd, w3_r[...]) + b3_r[...])
    s = jnp.sum(_dot(nd, wo_r[...]))
    i = pl.program_id(0)

    @pl.when(i == 0)
    def _():
        out_o[...] = jnp.reshape(s, (1, 1))

    @pl.when(i != 0)
    def _():
        out_o[...] = out_o[...] + jnp.reshape(s, (1, 1))


def _sds(shape):
    return jax.ShapeDtypeStruct(shape, _F32)


def kernel(z, pos, edge_index, idx_kj, idx_ji, params):
    E = edge_index.shape[1]
    T = idx_kj.shape[0]
    N = pos.shape[0]
    be = _tile(E)
    bt = _tile(T)
    bn = _tile(N, 1024)
    ge, gt, gn = E // be, T // bt, N // bn

    src = edge_index[0]
    dst = edge_index[1]
    ps = jnp.take(pos, src, axis=0)
    pd = jnp.take(pos, dst, axis=0)
    emb_pad = jnp.concatenate(
        [params['emb'], jnp.zeros((96 - params['emb'].shape[0], H), _F32)],
        axis=0)
    zs = jnp.take(z, src, axis=0).astype(_F32).reshape(E, 1)
    zd = jnp.take(z, dst, axis=0).astype(_F32).reshape(E, 1)
    wlin = params['emb_lin_W']
    w1, w2, w3 = wlin[:H], wlin[H:2 * H], wlin[2 * H:]
    b_emb = params['emb_lin_b'].reshape(1, H)
    outs = params['outs']
    blocks = params['blocks']

    dvec, d, rbf, x, g = pl.pallas_call(
        _embed_body,
        grid=(ge,),
        in_specs=[_rows(be, 3), _rows(be, 3), _rows(be, 1), _rows(be, 1),
                  _bcast((96, H)), _bcast((NR, H)), _bcast((H, H)),
                  _bcast((H, H)), _bcast((H, H)), _bcast((1, H)),
                  _bcast((NR, H))],
        out_specs=[_rows(be, 3), _rows(be, 1), _rows(be, NR), _rows(be, H),
                   _rows(be, H)],
        out_shape=[_sds((E, 3)), _sds((E, 1)), _sds((E, NR)), _sds((E, H)),
                   _sds((E, H))],
    )(ps, pd, zs, zd, emb_pad, params['emb_rbf'], w1, w2, w3, b_emb,
      outs[0]['lin_rbf'])

    ba = _tile(E, 1024)
    gb = E // ba
    ch = 128
    p_tri = jnp.argsort(idx_ji)
    sji = jnp.take(idx_ji, p_tri)
    skj = jnp.take(idx_kj, p_tri)
    starts = jnp.searchsorted(sji, jnp.arange(0, E + 1, ba)).astype(jnp.int32)
    lid_pad = jnp.concatenate(
        [(sji % ba).astype(_F32).reshape(-1, 1),
         jnp.full((ch, 1), -1.0, _F32)], axis=0)
    tp = T + ch

    v1 = jnp.take(dvec, sji, axis=0)
    v2 = jnp.take(dvec, skj, axis=0)
    dk = jnp.take(d, skj, axis=0)
    sbf = pl.pallas_call(
        _trigeom_body,
        grid=(gt,),
        in_specs=[_rows(bt, 3), _rows(bt, 3), _rows(bt, 1)],
        out_specs=_rows(bt, NS * NR),
        out_shape=_sds((T, NS * NR)),
    )(v1, v2, dk)

    def run_out(po, g_edge):
        node = jax.ops.segment_sum(g_edge, dst, num_segments=N)
        return pl.pallas_call(
            _out_node_body,
            grid=(gn,),
            in_specs=[_rows(bn, H), _bcast((H, H)), _bcast((1, H)),
                      _bcast((H, H)), _bcast((1, H)), _bcast((H, H)),
                      _bcast((1, H)), _bcast((H, 1))],
            out_specs=_bcast((1, 1)),
            out_shape=_sds((1, 1)),
        )(node, po['l1_W'], po['l1_b'].reshape(1, H), po['l2_W'],
          po['l2_b'].reshape(1, H), po['l3_W'], po['l3_b'].reshape(1, H),
          po['out_W'])

    P = run_out(outs[0], g)

    for bi, pb in enumerate(blocks):
        xji, xkj = pl.pallas_call(
            _stage_a_body,
            grid=(ge,),
            in_specs=[_rows(be, H), _rows(be, NR), _bcast((NR, H)),
                      _bcast((H, H)), _bcast((1, H)), _bcast((H, H)),
                      _bcast((1, H))],
            out_specs=[_rows(be, H), _rows(be, H)],
            out_shape=[_sds((E, H)), _sds((E, H))],
        )(x, rbf, pb['lin_rbf'], pb['kj_W'], pb['kj_b'].reshape(1, H),
          pb['ji_W'], pb['ji_b'].reshape(1, H))

        xk = jnp.take(xkj, skj, axis=0)
        y = pl.pallas_call(
            _tri_body,
            grid=(gt,),
            in_specs=[_rows(bt, NS * NR), _rows(bt, H),
                      _bcast((NS * NR, NB)), _bcast((NB, H, H))],
            out_specs=_rows(bt, H),
            out_shape=_sds((T, H)),
        )(sbf, xk, pb['lin_sbf'], pb['Wbil'])

        y_pad = jnp.concatenate([y, jnp.zeros((ch, H), _F32)], axis=0)
        x, g = pl.pallas_call(
            functools.partial(_stage_b_scatter_body, ba=ba, ch=ch),
            grid_spec=pltpu.PrefetchScalarGridSpec(
                num_scalar_prefetch=1,
                grid=(gb,),
                in_specs=[_bcast((tp, H)), _bcast((tp, 1)), _rows(ba, H),
                          _rows(ba, H), _rows(ba, NR), _bcast((H, H)),
                          _bcast((1, H)), _bcast((H, H)), _bcast((1, H)),
                          _bcast((H, H)), _bcast((1, H)), _bcast((H, H)),
                          _bcast((1, H)), _bcast((NR, H))],
                out_specs=[_rows(ba, H), _rows(ba, H)],
                scratch_shapes=[pltpu.VMEM((ba, H), _F32)],
            ),
            out_shape=[_sds((E, H)), _sds((E, H))],
        )(starts, y_pad, lid_pad, xji, x, rbf, pb['bs_W'],
          pb['bs_b'].reshape(1, H), pb['lin_W'], pb['lin_b'].reshape(1, H),
          pb['as1_W'], pb['as1_b'].reshape(1, H), pb['as2_W'],
          pb['as2_b'].reshape(1, H), outs[bi + 1]['lin_rbf'])

        P = P + run_out(outs[bi + 1], g)

    return P.reshape((1,))


# speedup vs baseline: 1.1003x; 1.0070x over previous
import functools

import numpy as np
import jax
import jax.numpy as jnp
from jax.experimental import pallas as pl
from jax.experimental.pallas import tpu as pltpu

H = 128
NR = 6
NS = 7
NB = 8
CUTOFF = 5.0
_F32 = jnp.float32


def _tile(n, cap=2048):
    c = min(cap, n)
    c -= c % 8
    while c >= 8:
        if n % c == 0:
            return c
        c -= 8
    return n


def _swish(v):
    return v * jax.nn.sigmoid(v)


def _dot(a, b):
    return jnp.dot(a, b, preferred_element_type=_F32)


def _iota_row(k):
    return jax.lax.broadcasted_iota(jnp.int32, (1, k), 1).astype(_F32)


def _rows(bsz, width):
    return pl.BlockSpec((bsz, width), lambda i, *_: (i, 0))


def _bcast(shape):
    nd = len(shape)
    return pl.BlockSpec(shape, lambda i, *_, _nd=nd: (0,) * _nd)


def _embed_body(ps_r, pd_r, zs_r, zd_r, emb_r, wr_r, w1_r, w2_r, w3_r, b_r,
                wg_r, dvec_o, d_o, rbf_o, x_o, g_o):
    dvec = pd_r[...] - ps_r[...]
    d = jnp.sqrt(jnp.sum(dvec * dvec, axis=1, keepdims=True) + 1e-12)
    n = _iota_row(NR) + 1.0
    rbf = jnp.exp(-(d / CUTOFF) ** 2) * jnp.sin(d * (n * (np.pi / CUTOFF))) / (d + 1e-6)
    hrbf = _swish(_dot(rbf, wr_r[...]))
    zcols = jax.lax.broadcasted_iota(jnp.int32, (zs_r.shape[0], 96), 1).astype(_F32)
    es = _dot(jnp.where(zs_r[...] == zcols, 1.0, 0.0), emb_r[...])
    ed = _dot(jnp.where(zd_r[...] == zcols, 1.0, 0.0), emb_r[...])
    x = _swish(_dot(es, w1_r[...]) + _dot(ed, w2_r[...])
               + _dot(hrbf, w3_r[...]) + b_r[...])
    dvec_o[...] = dvec
    d_o[...] = d
    rbf_o[...] = rbf
    x_o[...] = x
    g_o[...] = _dot(rbf, wg_r[...]) * x


def _trigeom_body(v1_r, v2_r, dk_r, sbf_o):
    v1 = v1_r[...]
    v2 = -v2_r[...]
    num = jnp.sum(v1 * v2, axis=1, keepdims=True)
    n1 = jnp.sqrt(jnp.sum(v1 * v1, axis=1, keepdims=True))
    n2 = jnp.sqrt(jnp.sum(v2 * v2, axis=1, keepdims=True))
    c = jnp.clip(num / (n1 * n2 + 1e-6), -0.999999, 0.999999)
    dk = dk_r[...]
    n = _iota_row(NR) + 1.0
    rad = jnp.sin(dk * (n * (np.pi / CUTOFF))) / (dk + 1e-6)
    parts = [rad]
    t_prev = jnp.ones_like(c)
    t_cur = c
    parts.append(rad * t_cur)
    for _ in range(2, NS):
        t_prev, t_cur = t_cur, 2.0 * c * t_cur - t_prev
        parts.append(rad * t_cur)
    sbf_o[...] = jnp.concatenate(parts, axis=1)


def _stage_a_body(x_r, rbf_r, wlr_r, wkj_r, bkj_r, wji_r, bji_r,
                  xji_o, xkj_o):
    x = x_r[...]
    rbf_h = _dot(rbf_r[...], wlr_r[...])
    xji_o[...] = _swish(_dot(x, wji_r[...]) + bji_r[...])
    xkj_o[...] = _swish(_dot(x, wkj_r[...]) + bkj_r[...]) * rbf_h


def _tri_body(sbf_r, xk_r, ws_r, wbil_r, y_o):
    sbf_h = _dot(sbf_r[...], ws_r[...])
    xk = xk_r[...]
    acc = sbf_h[:, 0:1] * _dot(xk, wbil_r[0])
    for b in range(1, NB):
        acc = acc + sbf_h[:, b:b + 1] * _dot(xk, wbil_r[b])
    y_o[...] = acc


def _stage_b_scatter_body(starts_r, y_r, lid_r, xji_r, x_r, rbf_r, wbs_r,
                          bbs_r, wl_r, bl_r, wa1_r, ba1_r, wa2_r, ba2_r,
                          wg_r, x_o, g_o, acc_r, *, ba, ch):
    i = pl.program_id(0)
    s = starts_r[i]
    e = starts_r[i + 1]
    s0 = (s // ch) * ch
    nch = (e - s0 + (ch - 1)) // ch
    acc_r[...] = jnp.zeros_like(acc_r)

    col = jax.lax.broadcasted_iota(jnp.int32, (ch, ba), 1).astype(_F32)
    row = jax.lax.broadcasted_iota(jnp.int32, (ch, 1), 0)

    def body(k, carry):
        base = s0 + k * ch
        yc = y_r[pl.ds(base, ch), :]
        lc = lid_r[pl.ds(base, ch), :]
        gidx = row + base
        oh = jnp.where((lc == col) & (gidx >= s) & (gidx < e), 1.0, 0.0)
        acc_r[...] += jax.lax.dot_general(
            oh, yc, (((0,), (0,)), ((), ())), preferred_element_type=_F32)
        return carry

    jax.lax.fori_loop(0, nch, body, 0)

    h = xji_r[...] + acc_r[...]
    h = h + _swish(_dot(h, wbs_r[...]) + bbs_r[...])
    h = _swish(_dot(h, wl_r[...]) + bl_r[...]) + x_r[...]
    h = h + _swish(_dot(h, wa1_r[...]) + ba1_r[...])
    h = h + _swish(_dot(h, wa2_r[...]) + ba2_r[...])
    x_o[...] = h
    g_o[...] = _dot(rbf_r[...], wg_r[...]) * h


def _stage_b_body(xji_r, agg_r, x_r, rbf_r, wbs_r, bbs_r, wl_r, bl_r,
                  wa1_r, ba1_r, wa2_r, ba2_r, wg_r, x_o, g_o):
    h = xji_r[...] + agg_r[...]
    h = h + _swish(_dot(h, wbs_r[...]) + bbs_r[...])
    h = _swish(_dot(h, wl_r[...]) + bl_r[...]) + x_r[...]
    h = h + _swish(_dot(h, wa1_r[...]) + ba1_r[...])
    h = h + _swish(_dot(h, wa2_r[...]) + ba2_r[...])
    x_o[...] = h
    g_o[...] = _dot(rbf_r[...], wg_r[...]) * h


def _out_node_body(node_r, w1_r, b1_r, w2_r, b2_r, w3_r, b3_r, wo_r, out_o):
    nd = _swish(_dot(node_r[...], w1_r[...]) + b1_r[...])
    nd = _swish(_dot(nd, w2_r[...]) + b2_r[...])
    nd = _swish(_dot(nd, w3_r[...]) + b3_r[...])
    s = jnp.sum(_dot(nd, wo_r[...]))
    i = pl.program_id(0)

    @pl.when(i == 0)
    def _():
        out_o[...] = jnp.reshape(s, (1, 1))

    @pl.when(i != 0)
    def _():
        out_o[...] = out_o[...] + jnp.reshape(s, (1, 1))


def _sds(shape):
    return jax.ShapeDtypeStruct(shape, _F32)


def kernel(z, pos, edge_index, idx_kj, idx_ji, params):
    E = edge_index.shape[1]
    T = idx_kj.shape[0]
    N = pos.shape[0]
    be = _tile(E)
    bt = _tile(T)
    bn = _tile(N, 1024)
    ge, gt, gn = E // be, T // bt, N // bn

    src = edge_index[0]
    dst = edge_index[1]
    ps = jnp.take(pos, src, axis=0)
    pd = jnp.take(pos, dst, axis=0)
    emb_pad = jnp.concatenate(
        [params['emb'], jnp.zeros((96 - params['emb'].shape[0], H), _F32)],
        axis=0)
    zs = jnp.take(z, src, axis=0).astype(_F32).reshape(E, 1)
    zd = jnp.take(z, dst, axis=0).astype(_F32).reshape(E, 1)
    wlin = params['emb_lin_W']
    w1, w2, w3 = wlin[:H], wlin[H:2 * H], wlin[2 * H:]
    b_emb = params['emb_lin_b'].reshape(1, H)
    outs = params['outs']
    blocks = params['blocks']

    dvec, d, rbf, x, g = pl.pallas_call(
        _embed_body,
        grid=(ge,),
        in_specs=[_rows(be, 3), _rows(be, 3), _rows(be, 1), _rows(be, 1),
                  _bcast((96, H)), _bcast((NR, H)), _bcast((H, H)),
                  _bcast((H, H)), _bcast((H, H)), _bcast((1, H)),
                  _bcast((NR, H))],
        out_specs=[_rows(be, 3), _rows(be, 1), _rows(be, NR), _rows(be, H),
                   _rows(be, H)],
        out_shape=[_sds((E, 3)), _sds((E, 1)), _sds((E, NR)), _sds((E, H)),
                   _sds((E, H))],
    )(ps, pd, zs, zd, emb_pad, params['emb_rbf'], w1, w2, w3, b_emb,
      outs[0]['lin_rbf'])

    ba = _tile(E, 1024)
    gb = E // ba
    ch = 256
    p_tri = jnp.argsort(idx_ji)
    sji = jnp.take(idx_ji, p_tri)
    skj = jnp.take(idx_kj, p_tri)
    starts = jnp.searchsorted(sji, jnp.arange(0, E + 1, ba)).astype(jnp.int32)
    lid_pad = jnp.concatenate(
        [(sji % ba).astype(_F32).reshape(-1, 1),
         jnp.full((ch, 1), -1.0, _F32)], axis=0)
    tp = T + ch

    v1 = jnp.take(dvec, sji, axis=0)
    v2 = jnp.take(dvec, skj, axis=0)
    dk = jnp.take(d, skj, axis=0)
    sbf = pl.pallas_call(
        _trigeom_body,
        grid=(gt,),
        in_specs=[_rows(bt, 3), _rows(bt, 3), _rows(bt, 1)],
        out_specs=_rows(bt, NS * NR),
        out_shape=_sds((T, NS * NR)),
    )(v1, v2, dk)

    def run_out(po, g_edge):
        node = jax.ops.segment_sum(g_edge, dst, num_segments=N)
        return pl.pallas_call(
            _out_node_body,
            grid=(gn,),
            in_specs=[_rows(bn, H), _bcast((H, H)), _bcast((1, H)),
                      _bcast((H, H)), _bcast((1, H)), _bcast((H, H)),
                      _bcast((1, H)), _bcast((H, 1))],
            out_specs=_bcast((1, 1)),
            out_shape=_sds((1, 1)),
        )(node, po['l1_W'], po['l1_b'].reshape(1, H), po['l2_W'],
          po['l2_b'].reshape(1, H), po['l3_W'], po['l3_b'].reshape(1, H),
          po['out_W'])

    P = run_out(outs[0], g)

    for bi, pb in enumerate(blocks):
        xji, xkj = pl.pallas_call(
            _stage_a_body,
            grid=(ge,),
            in_specs=[_rows(be, H), _rows(be, NR), _bcast((NR, H)),
                      _bcast((H, H)), _bcast((1, H)), _bcast((H, H)),
                      _bcast((1, H))],
            out_specs=[_rows(be, H), _rows(be, H)],
            out_shape=[_sds((E, H)), _sds((E, H))],
        )(x, rbf, pb['lin_rbf'], pb['kj_W'], pb['kj_b'].reshape(1, H),
          pb['ji_W'], pb['ji_b'].reshape(1, H))

        xk = jnp.take(xkj, skj, axis=0)
        y = pl.pallas_call(
            _tri_body,
            grid=(gt,),
            in_specs=[_rows(bt, NS * NR), _rows(bt, H),
                      _bcast((NS * NR, NB)), _bcast((NB, H, H))],
            out_specs=_rows(bt, H),
            out_shape=_sds((T, H)),
        )(sbf, xk, pb['lin_sbf'], pb['Wbil'])

        y_pad = jnp.concatenate([y, jnp.zeros((ch, H), _F32)], axis=0)
        x, g = pl.pallas_call(
            functools.partial(_stage_b_scatter_body, ba=ba, ch=ch),
            grid_spec=pltpu.PrefetchScalarGridSpec(
                num_scalar_prefetch=1,
                grid=(gb,),
                in_specs=[_bcast((tp, H)), _bcast((tp, 1)), _rows(ba, H),
                          _rows(ba, H), _rows(ba, NR), _bcast((H, H)),
                          _bcast((1, H)), _bcast((H, H)), _bcast((1, H)),
                          _bcast((H, H)), _bcast((1, H)), _bcast((H, H)),
                          _bcast((1, H)), _bcast((NR, H))],
                out_specs=[_rows(ba, H), _rows(ba, H)],
                scratch_shapes=[pltpu.VMEM((ba, H), _F32)],
            ),
            out_shape=[_sds((E, H)), _sds((E, H))],
        )(starts, y_pad, lid_pad, xji, x, rbf, pb['bs_W'],
          pb['bs_b'].reshape(1, H), pb['lin_W'], pb['lin_b'].reshape(1, H),
          pb['as1_W'], pb['as1_b'].reshape(1, H), pb['as2_W'],
          pb['as2_b'].reshape(1, H), outs[bi + 1]['lin_rbf'])

        P = P + run_out(outs[bi + 1], g)

    return P.reshape((1,))


# speedup vs baseline: 1.1154x; 1.0137x over previous
import functools

import numpy as np
import jax
import jax.numpy as jnp
from jax.experimental import pallas as pl
from jax.experimental.pallas import tpu as pltpu

H = 128
NR = 6
NS = 7
NB = 8
CUTOFF = 5.0
_F32 = jnp.float32


def _tile(n, cap=2048):
    c = min(cap, n)
    c -= c % 8
    while c >= 8:
        if n % c == 0:
            return c
        c -= 8
    return n


def _swish(v):
    return v * jax.nn.sigmoid(v)


def _dot(a, b):
    return jnp.dot(a, b, preferred_element_type=_F32)


def _iota_row(k):
    return jax.lax.broadcasted_iota(jnp.int32, (1, k), 1).astype(_F32)


def _rows(bsz, width):
    return pl.BlockSpec((bsz, width), lambda i, *_: (i, 0))


def _bcast(shape):
    nd = len(shape)
    return pl.BlockSpec(shape, lambda i, *_, _nd=nd: (0,) * _nd)


def _embed_body(ps_r, pd_r, zs_r, zd_r, emb_r, wr_r, w1_r, w2_r, w3_r, b_r,
                wg_r, dvec_o, d_o, rbf_o, x_o, g_o):
    dvec = pd_r[...] - ps_r[...]
    d = jnp.sqrt(jnp.sum(dvec * dvec, axis=1, keepdims=True) + 1e-12)
    n = _iota_row(NR) + 1.0
    rbf = jnp.exp(-(d / CUTOFF) ** 2) * jnp.sin(d * (n * (np.pi / CUTOFF))) / (d + 1e-6)
    hrbf = _swish(_dot(rbf, wr_r[...]))
    zcols = jax.lax.broadcasted_iota(jnp.int32, (zs_r.shape[0], 96), 1).astype(_F32)
    es = _dot(jnp.where(zs_r[...] == zcols, 1.0, 0.0), emb_r[...])
    ed = _dot(jnp.where(zd_r[...] == zcols, 1.0, 0.0), emb_r[...])
    x = _swish(_dot(es, w1_r[...]) + _dot(ed, w2_r[...])
               + _dot(hrbf, w3_r[...]) + b_r[...])
    dvec_o[...] = dvec
    d_o[...] = d
    rbf_o[...] = rbf
    x_o[...] = x
    g_o[...] = _dot(rbf, wg_r[...]) * x


def _trigeom_body(v1_r, v2_r, dk_r, sbf_o):
    v1 = v1_r[...]
    v2 = -v2_r[...]
    num = jnp.sum(v1 * v2, axis=1, keepdims=True)
    n1 = jnp.sqrt(jnp.sum(v1 * v1, axis=1, keepdims=True))
    n2 = jnp.sqrt(jnp.sum(v2 * v2, axis=1, keepdims=True))
    c = jnp.clip(num / (n1 * n2 + 1e-6), -0.999999, 0.999999)
    dk = dk_r[...]
    n = _iota_row(NR) + 1.0
    rad = jnp.sin(dk * (n * (np.pi / CUTOFF))) / (dk + 1e-6)
    parts = [rad]
    t_prev = jnp.ones_like(c)
    t_cur = c
    parts.append(rad * t_cur)
    for _ in range(2, NS):
        t_prev, t_cur = t_cur, 2.0 * c * t_cur - t_prev
        parts.append(rad * t_cur)
    sbf_o[...] = jnp.concatenate(parts, axis=1)


def _stage_a_body(x_r, rbf_r, wlr_r, wkj_r, bkj_r, wji_r, bji_r,
                  xji_o, xkj_o):
    x = x_r[...]
    rbf_h = _dot(rbf_r[...], wlr_r[...])
    xji_o[...] = _swish(_dot(x, wji_r[...]) + bji_r[...])
    xkj_o[...] = _swish(_dot(x, wkj_r[...]) + bkj_r[...]) * rbf_h


def _tri_body(sbf_r, xk_r, ws_r, wbil_r, y_o):
    sbf_h = _dot(sbf_r[...], ws_r[...])
    xk = xk_r[...]
    acc = sbf_h[:, 0:1] * _dot(xk, wbil_r[0])
    for b in range(1, NB):
        acc = acc + sbf_h[:, b:b + 1] * _dot(xk, wbil_r[b])
    y_o[...] = acc


def _stage_b_scatter_body(starts_r, y_r, lid_r, xji_r, x_r, rbf_r, wbs_r,
                          bbs_r, wl_r, bl_r, wa1_r, ba1_r, wa2_r, ba2_r,
                          wg_r, x_o, g_o, acc_r, *, ba, ch):
    i = pl.program_id(0)
    s = starts_r[i]
    e = starts_r[i + 1]
    s0 = (s // ch) * ch
    nch = (e - s0 + (ch - 1)) // ch
    acc_r[...] = jnp.zeros_like(acc_r)

    col = jax.lax.broadcasted_iota(jnp.int32, (ch, ba), 1).astype(_F32)
    row = jax.lax.broadcasted_iota(jnp.int32, (ch, 1), 0)

    def body(k, carry):
        base = s0 + k * ch
        yc = y_r[pl.ds(base, ch), :]
        lc = lid_r[pl.ds(base, ch), :]
        gidx = row + base
        oh = jnp.where((lc == col) & (gidx >= s) & (gidx < e), 1.0, 0.0)
        acc_r[...] += jax.lax.dot_general(
            oh, yc, (((0,), (0,)), ((), ())), preferred_element_type=_F32)
        return carry

    jax.lax.fori_loop(0, nch, body, 0)

    h = xji_r[...] + acc_r[...]
    h = h + _swish(_dot(h, wbs_r[...]) + bbs_r[...])
    h = _swish(_dot(h, wl_r[...]) + bl_r[...]) + x_r[...]
    h = h + _swish(_dot(h, wa1_r[...]) + ba1_r[...])
    h = h + _swish(_dot(h, wa2_r[...]) + ba2_r[...])
    x_o[...] = h
    g_o[...] = _dot(rbf_r[...], wg_r[...]) * h


def _stage_b_body(xji_r, agg_r, x_r, rbf_r, wbs_r, bbs_r, wl_r, bl_r,
                  wa1_r, ba1_r, wa2_r, ba2_r, wg_r, x_o, g_o):
    h = xji_r[...] + agg_r[...]
    h = h + _swish(_dot(h, wbs_r[...]) + bbs_r[...])
    h = _swish(_dot(h, wl_r[...]) + bl_r[...]) + x_r[...]
    h = h + _swish(_dot(h, wa1_r[...]) + ba1_r[...])
    h = h + _swish(_dot(h, wa2_r[...]) + ba2_r[...])
    x_o[...] = h
    g_o[...] = _dot(rbf_r[...], wg_r[...]) * h


def _out_node_body(node_r, w1_r, b1_r, w2_r, b2_r, w3_r, b3_r, wo_r, out_o):
    nd = _swish(_dot(node_r[...], w1_r[...]) + b1_r[...])
    nd = _swish(_dot(nd, w2_r[...]) + b2_r[...])
    nd = _swish(_dot(nd, w3_r[...]) + b3_r[...])
    s = jnp.sum(_dot(nd, wo_r[...]))
    i = pl.program_id(0)

    @pl.when(i == 0)
    def _():
        out_o[...] = jnp.reshape(s, (1, 1))

    @pl.when(i != 0)
    def _():
        out_o[...] = out_o[...] + jnp.reshape(s, (1, 1))


def _sds(shape):
    return jax.ShapeDtypeStruct(shape, _F32)


def kernel(z, pos, edge_index, idx_kj, idx_ji, params):
    E = edge_index.shape[1]
    T = idx_kj.shape[0]
    N = pos.shape[0]
    be = _tile(E, 4096)
    bt = _tile(T, 4096)
    bn = _tile(N, 1024)
    ge, gt, gn = E // be, T // bt, N // bn

    src = edge_index[0]
    dst = edge_index[1]
    ps = jnp.take(pos, src, axis=0)
    pd = jnp.take(pos, dst, axis=0)
    emb_pad = jnp.concatenate(
        [params['emb'], jnp.zeros((96 - params['emb'].shape[0], H), _F32)],
        axis=0)
    zs = jnp.take(z, src, axis=0).astype(_F32).reshape(E, 1)
    zd = jnp.take(z, dst, axis=0).astype(_F32).reshape(E, 1)
    wlin = params['emb_lin_W']
    w1, w2, w3 = wlin[:H], wlin[H:2 * H], wlin[2 * H:]
    b_emb = params['emb_lin_b'].reshape(1, H)
    outs = params['outs']
    blocks = params['blocks']

    dvec, d, rbf, x, g = pl.pallas_call(
        _embed_body,
        grid=(ge,),
        in_specs=[_rows(be, 3), _rows(be, 3), _rows(be, 1), _rows(be, 1),
                  _bcast((96, H)), _bcast((NR, H)), _bcast((H, H)),
                  _bcast((H, H)), _bcast((H, H)), _bcast((1, H)),
                  _bcast((NR, H))],
        out_specs=[_rows(be, 3), _rows(be, 1), _rows(be, NR), _rows(be, H),
                   _rows(be, H)],
        out_shape=[_sds((E, 3)), _sds((E, 1)), _sds((E, NR)), _sds((E, H)),
                   _sds((E, H))],
    )(ps, pd, zs, zd, emb_pad, params['emb_rbf'], w1, w2, w3, b_emb,
      outs[0]['lin_rbf'])

    ba = _tile(E, 1024)
    gb = E // ba
    ch = 256
    p_tri = jnp.argsort(idx_ji)
    sji = jnp.take(idx_ji, p_tri)
    skj = jnp.take(idx_kj, p_tri)
    starts = jnp.searchsorted(sji, jnp.arange(0, E + 1, ba)).astype(jnp.int32)
    lid_pad = jnp.concatenate(
        [(sji % ba).astype(_F32).reshape(-1, 1),
         jnp.full((ch, 1), -1.0, _F32)], axis=0)
    tp = T + ch

    v1 = jnp.take(dvec, sji, axis=0)
    v2 = jnp.take(dvec, skj, axis=0)
    dk = jnp.take(d, skj, axis=0)
    sbf = pl.pallas_call(
        _trigeom_body,
        grid=(gt,),
        in_specs=[_rows(bt, 3), _rows(bt, 3), _rows(bt, 1)],
        out_specs=_rows(bt, NS * NR),
        out_shape=_sds((T, NS * NR)),
    )(v1, v2, dk)

    def run_out(po, g_edge):
        node = jax.ops.segment_sum(g_edge, dst, num_segments=N)
        return pl.pallas_call(
            _out_node_body,
            grid=(gn,),
            in_specs=[_rows(bn, H), _bcast((H, H)), _bcast((1, H)),
                      _bcast((H, H)), _bcast((1, H)), _bcast((H, H)),
                      _bcast((1, H)), _bcast((H, 1))],
            out_specs=_bcast((1, 1)),
            out_shape=_sds((1, 1)),
        )(node, po['l1_W'], po['l1_b'].reshape(1, H), po['l2_W'],
          po['l2_b'].reshape(1, H), po['l3_W'], po['l3_b'].reshape(1, H),
          po['out_W'])

    P = run_out(outs[0], g)

    for bi, pb in enumerate(blocks):
        xji, xkj = pl.pallas_call(
            _stage_a_body,
            grid=(ge,),
            in_specs=[_rows(be, H), _rows(be, NR), _bcast((NR, H)),
                      _bcast((H, H)), _bcast((1, H)), _bcast((H, H)),
                      _bcast((1, H))],
            out_specs=[_rows(be, H), _rows(be, H)],
            out_shape=[_sds((E, H)), _sds((E, H))],
        )(x, rbf, pb['lin_rbf'], pb['kj_W'], pb['kj_b'].reshape(1, H),
          pb['ji_W'], pb['ji_b'].reshape(1, H))

        xk = jnp.take(xkj, skj, axis=0)
        y = pl.pallas_call(
            _tri_body,
            grid=(gt,),
            in_specs=[_rows(bt, NS * NR), _rows(bt, H),
                      _bcast((NS * NR, NB)), _bcast((NB, H, H))],
            out_specs=_rows(bt, H),
            out_shape=_sds((T, H)),
        )(sbf, xk, pb['lin_sbf'], pb['Wbil'])

        y_pad = jnp.concatenate([y, jnp.zeros((ch, H), _F32)], axis=0)
        x, g = pl.pallas_call(
            functools.partial(_stage_b_scatter_body, ba=ba, ch=ch),
            grid_spec=pltpu.PrefetchScalarGridSpec(
                num_scalar_prefetch=1,
                grid=(gb,),
                in_specs=[_bcast((tp, H)), _bcast((tp, 1)), _rows(ba, H),
                          _rows(ba, H), _rows(ba, NR), _bcast((H, H)),
                          _bcast((1, H)), _bcast((H, H)), _bcast((1, H)),
                          _bcast((H, H)), _bcast((1, H)), _bcast((H, H)),
                          _bcast((1, H)), _bcast((NR, H))],
                out_specs=[_rows(ba, H), _rows(ba, H)],
                scratch_shapes=[pltpu.VMEM((ba, H), _F32)],
            ),
            out_shape=[_sds((E, H)), _sds((E, H))],
        )(starts, y_pad, lid_pad, xji, x, rbf, pb['bs_W'],
          pb['bs_b'].reshape(1, H), pb['lin_W'], pb['lin_b'].reshape(1, H),
          pb['as1_W'], pb['as1_b'].reshape(1, H), pb['as2_W'],
          pb['as2_b'].reshape(1, H), outs[bi + 1]['lin_rbf'])

        P = P + run_out(outs[bi + 1], g)

    return P.reshape((1,))
